# baseline (speedup 1.0000x reference)
import functools

import jax
import jax.numpy as jnp
from jax import lax
from jax.experimental import pallas as pl
from jax.experimental.pallas import tpu as pltpu

_D = _H = _W = 128
_YC = 8
_NT = _H // _YC
_ZU = 3
_SYU = 5


def _tile_kernel(bnd_ref, fwd_ref, bwd_ref, out_ref):
    z = pl.program_id(1)
    zf = z.astype(jnp.float32)
    iota_si = lax.broadcasted_iota(jnp.int32, (_YC, _W), 0)
    iota_s = iota_si.astype(jnp.float32)
    iota_l = lax.broadcasted_iota(jnp.int32, (_YC, _W), 1).astype(jnp.float32)

    def tile_body(yc, tot):
        ys = pl.multiple_of(yc * _YC, _YC)
        fz = fwd_ref[0, 0, 0, pl.ds(ys, _YC), :]
        fy = fwd_ref[0, 1, 0, pl.ds(ys, _YC), :]
        fx = fwd_ref[0, 2, 0, pl.ds(ys, _YC), :]

        cz = jnp.clip(zf + fz, -2.0, 129.0)
        cy = jnp.clip(iota_s + (yc * _YC).astype(jnp.float32) + fy, -2.0, 129.0)
        cx = jnp.clip(iota_l + fx, -2.0, 129.0)
        z0f = jnp.floor(cz)
        y0f = jnp.floor(cy)
        x0f = jnp.floor(cx)
        wz = cz - z0f
        wy = cy - y0f
        wx = cx - x0f
        z0 = z0f.astype(jnp.int32)
        y0 = y0f.astype(jnp.int32)
        x0 = x0f.astype(jnp.int32)

        x1 = x0 + 1
        wx0 = jnp.where((x0 >= 0) & (x0 < _W), 1.0 - wx, 0.0)
        wx1 = jnp.where((x1 >= 0) & (x1 < _W), wx, 0.0)
        x0c = jnp.clip(x0, 0, _W - 1)
        x1c = jnp.clip(x1, 0, _W - 1)

        wzA = jnp.where(z0 <= _D - 1, 1.0 - wz, 0.0)
        wzB = jnp.where(z0 <= _D - 2, wz, 0.0)
        wyA = jnp.where((y0 >= 0) & (y0 <= _H - 1), 1.0 - wy, 0.0)
        wyB = jnp.where((y0 >= -1) & (y0 <= _H - 2), wy, 0.0)
        dyv = y0 - (iota_si + ys)

        zlo = bnd_ref[0, 0, yc, 0]
        nzg = bnd_ref[0, 0, yc, 1]
        sylo = bnd_ref[0, 0, yc, 2]
        nsy = bnd_ref[0, 0, yc, 3]

        zero = jnp.zeros((_YC, _W), jnp.float32)

        def sy_body(jy, acc):
            sys_ = []
            for v in range(_SYU):
                sy = sylo + jy * _SYU + v
                yrow0 = ys + sy
                a0 = yrow0 >> 3
                r = yrow0 & 7
                b0 = pl.multiple_of(jnp.clip(a0, 0, _NT - 1) * 8, 8)
                b1 = pl.multiple_of(jnp.clip(a0 + 1, 0, _NT - 1) * 8, 8)
                idxr = (iota_si + r) & 7
                srcmask = iota_si >= r
                wyeff = (jnp.where(dyv == sy, wyA, 0.0)
                         + jnp.where(dyv == sy - 1, wyB, 0.0))
                sys_.append((b0, b1, idxr, srcmask, wyeff))

            def plane_group_body(j, acc):
                zi0 = zlo + j * _ZU
                out = list(acc)
                for u in range(_ZU):
                    zi = zi0 + u
                    zil = jnp.minimum(zi, _D - 1)
                    wzeff = (jnp.where(z0 == zi, wzA, 0.0)
                             + jnp.where(z0 == zi - 1, wzB, 0.0))
                    for b0, b1, idxr, srcmask, wyeff in sys_:
                        w2 = wzeff * wyeff
                        wx0e = w2 * wx0
                        wx1e = w2 * wx1
                        for c in range(3):
                            c0 = bwd_ref[0, c, zil, pl.ds(b0, 8), :]
                            c1 = bwd_ref[0, c, zil, pl.ds(b1, 8), :]
                            srow = jnp.take_along_axis(
                                jnp.where(srcmask, c0, c1), idxr, axis=0)
                            v0 = jnp.take_along_axis(srow, x0c, axis=1)
                            v1 = jnp.take_along_axis(srow, x1c, axis=1)
                            out[c] = out[c] + (v0 * wx0e + v1 * wx1e)
                return tuple(out)

            return lax.fori_loop(0, nzg, plane_group_body, acc)

        acc = lax.fori_loop(0, nsy, sy_body, (zero, zero, zero))
        sq = ((fz + acc[0]) ** 2 + (fy + acc[1]) ** 2 + (fx + acc[2]) ** 2)
        return tot + sq

    total = lax.fori_loop(0, _NT, tile_body, jnp.zeros((_YC, _W), jnp.float32))

    @pl.when(z == 0)
    def _():
        out_ref[0] = total

    @pl.when(z != 0)
    def _():
        out_ref[0] += total


@functools.partial(jax.jit, static_argnames=("interpret",))
def kernel(forward_disp, backward_disp, interpret=False):
    B = forward_disp.shape[0]

    zidx = lax.broadcasted_iota(jnp.float32, (1, _D, 1, 1, 1), 1)
    yidx = lax.broadcasted_iota(jnp.float32, (1, 1, 1, _YC, 1), 3)
    ycidx = lax.broadcasted_iota(jnp.float32, (1, 1, _NT, 1, 1), 2)
    fz = forward_disp[:, 0].reshape(B, _D, _NT, _YC, _W)
    fy = forward_disp[:, 1].reshape(B, _D, _NT, _YC, _W)
    z0 = jnp.floor(jnp.clip(fz + zidx, -2.0, 129.0)).astype(jnp.int32)
    y0 = jnp.floor(jnp.clip(fy + yidx + ycidx * _YC, -2.0, 129.0)).astype(jnp.int32)
    zlo = jnp.clip(jnp.min(z0, axis=(3, 4)), 0, _D - 1)
    zhi = jnp.clip(jnp.max(z0, axis=(3, 4)) + 1, 0, _D - 1)
    ngrp = (zhi - zlo + _ZU) // _ZU
    dy = y0 - (yidx + ycidx * _YC).astype(jnp.int32)
    sylo = jnp.min(dy, axis=(3, 4))
    nsy = (jnp.max(dy, axis=(3, 4)) - sylo + 1 + _SYU) // _SYU
    bounds = jnp.stack([zlo, ngrp, sylo, nsy], axis=-1)

    out = pl.pallas_call(
        _tile_kernel,
        grid=(B, _D),
        in_specs=[
            pl.BlockSpec((1, 1, _NT, 4), lambda b, z: (b, z, 0, 0),
                         memory_space=pltpu.SMEM),
            pl.BlockSpec((1, 3, 1, _H, _W), lambda b, z: (b, 0, z, 0, 0)),
            pl.BlockSpec((1, 3, _D, _H, _W), lambda b, z: (b, 0, 0, 0, 0)),
        ],
        out_specs=pl.BlockSpec((1, _YC, _W), lambda b, z: (b, 0, 0)),
        out_shape=jax.ShapeDtypeStruct((B, _YC, _W), jnp.float32),
        compiler_params=pltpu.CompilerParams(
            dimension_semantics=("parallel", "arbitrary")),
        interpret=interpret,
    )(bounds, forward_disp, backward_disp)

    n = forward_disp.size
    loss = jnp.sum(out) / jnp.float32(n)
    return jnp.nan_to_num(loss, nan=0.0, posinf=1000.0, neginf=0.0)


# speedup vs baseline: 1.0016x; 1.0016x over previous
import jax
import jax.numpy as jnp
from jax import lax
from jax.experimental import pallas as pl
from jax.experimental.pallas import tpu as pltpu

_D = _H = _W = 128
_YC = 8
_NT = _H // _YC
_ZU = 3
_SYU = 5


def _tile_kernel(bnd_ref, fwd_ref, bwd_ref, out_ref):
    z = pl.program_id(1)
    zf = z.astype(jnp.float32)
    iota_si = lax.broadcasted_iota(jnp.int32, (_YC, _W), 0)
    iota_s = iota_si.astype(jnp.float32)
    iota_l = lax.broadcasted_iota(jnp.int32, (_YC, _W), 1).astype(jnp.float32)

    def tile_body(yc, tot):
        ys = pl.multiple_of(yc * _YC, _YC)
        fz = fwd_ref[0, 0, 0, pl.ds(ys, _YC), :]
        fy = fwd_ref[0, 1, 0, pl.ds(ys, _YC), :]
        fx = fwd_ref[0, 2, 0, pl.ds(ys, _YC), :]

        cz = jnp.clip(zf + fz, -2.0, 129.0)
        cy = jnp.clip(iota_s + (yc * _YC).astype(jnp.float32) + fy, -2.0, 129.0)
        cx = jnp.clip(iota_l + fx, -2.0, 129.0)
        z0f = jnp.floor(cz)
        y0f = jnp.floor(cy)
        x0f = jnp.floor(cx)
        wz = cz - z0f
        wy = cy - y0f
        wx = cx - x0f
        z0 = z0f.astype(jnp.int32)
        y0 = y0f.astype(jnp.int32)
        x0 = x0f.astype(jnp.int32)

        x1 = x0 + 1
        wx0 = jnp.where((x0 >= 0) & (x0 < _W), 1.0 - wx, 0.0)
        wx1 = jnp.where((x1 >= 0) & (x1 < _W), wx, 0.0)
        x0c = jnp.clip(x0, 0, _W - 1)
        x1c = jnp.clip(x1, 0, _W - 1)

        wzA = jnp.where(z0 <= _D - 1, 1.0 - wz, 0.0)
        wzB = jnp.where(z0 <= _D - 2, wz, 0.0)
        wyA = jnp.where((y0 >= 0) & (y0 <= _H - 1), 1.0 - wy, 0.0)
        wyB = jnp.where((y0 >= -1) & (y0 <= _H - 2), wy, 0.0)
        dyv = y0 - (iota_si + ys)

        zlo = bnd_ref[0, 0, yc, 0]
        nzg = bnd_ref[0, 0, yc, 1]
        sylo = bnd_ref[0, 0, yc, 2]
        nsy = bnd_ref[0, 0, yc, 3]

        zero = jnp.zeros((_YC, _W), jnp.float32)

        def sy_body(jy, acc):
            sys_ = []
            for v in range(_SYU):
                sy = sylo + jy * _SYU + v
                yrow0 = ys + sy
                a0 = yrow0 >> 3
                r = yrow0 & 7
                b0 = pl.multiple_of(jnp.clip(a0, 0, _NT - 1) * 8, 8)
                b1 = pl.multiple_of(jnp.clip(a0 + 1, 0, _NT - 1) * 8, 8)
                idxr = (iota_si + r) & 7
                srcmask = iota_si >= r
                wyeff = (jnp.where(dyv == sy, wyA, 0.0)
                         + jnp.where(dyv == sy - 1, wyB, 0.0))
                sys_.append((b0, b1, idxr, srcmask, wyeff))

            def plane_group_body(j, acc):
                zi0 = zlo + j * _ZU
                out = list(acc)
                for u in range(_ZU):
                    zi = zi0 + u
                    zil = jnp.minimum(zi, _D - 1)
                    wzeff = (jnp.where(z0 == zi, wzA, 0.0)
                             + jnp.where(z0 == zi - 1, wzB, 0.0))
                    for b0, b1, idxr, srcmask, wyeff in sys_:
                        w2 = wzeff * wyeff
                        wx0e = w2 * wx0
                        wx1e = w2 * wx1
                        for c in range(3):
                            c0 = bwd_ref[0, c, zil, pl.ds(b0, 8), :]
                            c1 = bwd_ref[0, c, zil, pl.ds(b1, 8), :]
                            srow = jnp.take_along_axis(
                                jnp.where(srcmask, c0, c1), idxr, axis=0)
                            v0 = jnp.take_along_axis(srow, x0c, axis=1)
                            v1 = jnp.take_along_axis(srow, x1c, axis=1)
                            out[c] = out[c] + (v0 * wx0e + v1 * wx1e)
                return tuple(out)

            return lax.fori_loop(0, nzg, plane_group_body, acc)

        acc = lax.fori_loop(0, nsy, sy_body, (zero, zero, zero))
        sq = ((fz + acc[0]) ** 2 + (fy + acc[1]) ** 2 + (fx + acc[2]) ** 2)
        return tot + sq

    total = lax.fori_loop(0, _NT, tile_body, jnp.zeros((_YC, _W), jnp.float32))

    @pl.when(z == 0)
    def _():
        out_ref[0] = total

    @pl.when(z != 0)
    def _():
        out_ref[0] += total


@jax.jit
def kernel(forward_disp, backward_disp):
    B = forward_disp.shape[0]

    zidx = lax.broadcasted_iota(jnp.float32, (1, _D, 1, 1, 1), 1)
    yidx = lax.broadcasted_iota(jnp.float32, (1, 1, 1, _YC, 1), 3)
    ycidx = lax.broadcasted_iota(jnp.float32, (1, 1, _NT, 1, 1), 2)
    fz = forward_disp[:, 0].reshape(B, _D, _NT, _YC, _W)
    fy = forward_disp[:, 1].reshape(B, _D, _NT, _YC, _W)
    z0 = jnp.floor(jnp.clip(fz + zidx, -2.0, 129.0)).astype(jnp.int32)
    y0 = jnp.floor(jnp.clip(fy + yidx + ycidx * _YC, -2.0, 129.0)).astype(jnp.int32)
    zlo = jnp.clip(jnp.min(z0, axis=(3, 4)), 0, _D - 1)
    zhi = jnp.clip(jnp.max(z0, axis=(3, 4)) + 1, 0, _D - 1)
    ngrp = (zhi - zlo + _ZU) // _ZU
    dy = y0 - (yidx + ycidx * _YC).astype(jnp.int32)
    sylo = jnp.min(dy, axis=(3, 4))
    nsy = (jnp.max(dy, axis=(3, 4)) - sylo + 1 + _SYU) // _SYU
    bounds = jnp.stack([zlo, ngrp, sylo, nsy], axis=-1)

    out = pl.pallas_call(
        _tile_kernel,
        grid=(B, _D),
        in_specs=[
            pl.BlockSpec((1, 1, _NT, 4), lambda b, z: (b, z, 0, 0),
                         memory_space=pltpu.SMEM),
            pl.BlockSpec((1, 3, 1, _H, _W), lambda b, z: (b, 0, z, 0, 0)),
            pl.BlockSpec((1, 3, _D, _H, _W), lambda b, z: (b, 0, 0, 0, 0)),
        ],
        out_specs=pl.BlockSpec((1, _YC, _W), lambda b, z: (b, 0, 0)),
        out_shape=jax.ShapeDtypeStruct((B, _YC, _W), jnp.float32),
        compiler_params=pltpu.CompilerParams(
            dimension_semantics=("parallel", "arbitrary")),
    )(bounds, forward_disp, backward_disp)

    n = forward_disp.size
    loss = jnp.sum(out) / jnp.float32(n)
    return jnp.nan_to_num(loss, nan=0.0, posinf=1000.0, neginf=0.0)
